# Initial kernel scaffold; baseline (speedup 1.0000x reference)
import jax
import jax.numpy as jnp
from jax.experimental import pallas as pl


def kernel(edge_index, primary_emb, W_hops, gru_w_ih, gru_w_hh, gru_b_ih, gru_b_hh, mlp_w1, mlp_b1, mlp_w2, mlp_b2, mlp_w3, mlp_b3):
    raise NotImplementedError("write your pallas kernel here")


# speedup vs baseline: 11.6588x; 11.6588x over previous
import functools

import jax
import jax.numpy as jnp
from jax import lax
from jax.experimental import pallas as pl
from jax.experimental.pallas import tpu as pltpu
from jax.experimental.pallas import tpu_sc as plsc

CHUNK = 128


def _worker_id():
    c = lax.axis_index("c")
    s = lax.axis_index("s")
    return s * 2 + c, c, s


def _stripe_chunks(npt):
    out = []
    q = 0
    while q < npt:
        out.append((q, min(CHUNK, npt - q)))
        q += CHUNK
    return out


def _msg_body(h_hbm, ei_hbm, zeros_hbm, *refs, n_pad, e, with_deg):
    if with_deg:
        (out_hbm, degs_hbm,
         rows0, gsem0, rows1, gsem1,
         ib0, ism0, ib1, ism1, ib2, ism2, ib3, ism3,
         acc, ones_v, zbuf, accd_in, accd_out) = refs
    else:
        (out_hbm,
         rows0, gsem0, rows1, gsem1,
         ib0, ism0, ib1, ism1, ib2, ism2, ib3, ism3,
         acc) = refs
    wid, c, s = _worker_id()
    npt = n_pad // 16
    nch = e // CHUNK
    nj = (nch + 31) // 32
    nt = (nj + 3) // 4
    rows = [rows0, rows1]
    gsems = [gsem0, gsem1]
    pairs = [(ib0, ism0), (ib1, ism1), (ib2, ism2), (ib3, ism3)]

    pltpu.sync_copy(zeros_hbm.at[pl.ds(0, CHUNK)], rows0)
    for q, w in _stripe_chunks(npt):
        pltpu.sync_copy(rows0.at[pl.ds(0, w)], acc.at[pl.ds(s * npt + q, w)])
    if with_deg:
        for i in range(CHUNK // 16):
            ones_v[pl.ds(i * 16, 16)] = jnp.ones((16,), jnp.float32)
        zlen = zbuf.shape[0]
        for i in range(zlen // 16):
            zbuf[pl.ds(i * 16, 16)] = jnp.zeros((16,), jnp.float32)
        pltpu.sync_copy(zbuf.at[pl.ds(0, npt)], accd_in.at[pl.ds(s * npt, npt)])
        pltpu.sync_copy(zbuf.at[pl.ds(0, npt)], accd_out.at[pl.ds(s * npt, npt)])
    plsc.subcore_barrier()

    def idx_start(j, p):
        ib, ism = pairs[p]
        cid = j * 32 + wid

        @pl.when(cid < nch)
        def _():
            pltpu.async_copy(ei_hbm.at[:, pl.ds(cid * CHUNK, CHUNK)], ib, ism)

    def gather_start(j, b, p):
        ib, ism = pairs[p]
        cid = j * 32 + wid

        @pl.when(cid < nch)
        def _():
            pltpu.make_async_copy(
                ei_hbm.at[:, pl.ds(cid * CHUNK, CHUNK)], ib, ism).wait()
            pltpu.async_copy(h_hbm.at[ib.at[0]], rows[b], gsems[b])

    def finish(j, b, p):
        ib, ism = pairs[p]
        cid = j * 32 + wid

        @pl.when(cid < nch)
        def _():
            pltpu.make_async_copy(h_hbm.at[ib.at[0]], rows[b], gsems[b]).wait()
            pltpu.sync_copy(rows[b], acc.at[ib.at[1]], add=True)
            if with_deg:
                pltpu.sync_copy(ones_v, accd_in.at[ib.at[1]], add=True)
                pltpu.sync_copy(ones_v, accd_out.at[ib.at[0]], add=True)

    idx_start(0, 0)
    idx_start(1, 1)
    gather_start(0, 0, 0)
    gather_start(1, 1, 1)
    idx_start(2, 2)
    idx_start(3, 3)

    def body(t, carry):
        j = t * 4
        for off, b in ((0, 0), (1, 1), (2, 0), (3, 1)):
            p = off
            p_next = (off + 2) % 4
            finish(j + off, b, p)
            idx_start(j + off + 4, p)
            gather_start(j + off + 2, b, p_next)
        return carry

    lax.fori_loop(0, nt, body, 0)
    plsc.subcore_barrier()

    _writeback(acc, out_hbm, c, s, npt, rows, gsems)
    if with_deg:
        base_c = c * (2 * n_pad)
        pltpu.sync_copy(accd_in.at[pl.ds(s * npt, npt)], zbuf.at[pl.ds(0, npt)])
        pltpu.sync_copy(zbuf.at[pl.ds(0, npt)],
                        degs_hbm.at[pl.ds(base_c + s * npt, npt)])
        pltpu.sync_copy(accd_out.at[pl.ds(s * npt, npt)], zbuf.at[pl.ds(0, npt)])
        pltpu.sync_copy(zbuf.at[pl.ds(0, npt)],
                        degs_hbm.at[pl.ds(base_c + n_pad + s * npt, npt)])


def _writeback(acc, out_hbm, c, s, npt, rows, gsems):
    chunks = _stripe_chunks(npt)
    nb = len(rows)

    for i, (q, w) in enumerate(chunks):
        if i >= 2:
            continue
        pltpu.async_copy(acc.at[pl.ds(s * npt + q, w)],
                         rows[i % nb].at[pl.ds(0, w)], gsems[i % nb])
    for i, (q, w) in enumerate(chunks):
        b = i % nb
        pltpu.make_async_copy(acc.at[pl.ds(s * npt + q, w)],
                              rows[b].at[pl.ds(0, w)], gsems[b]).wait()
        pltpu.sync_copy(rows[b].at[pl.ds(0, w)], out_hbm.at[c, pl.ds(s * npt + q, w)])
        if i + 2 < len(chunks):
            q2, w2 = chunks[i + 2]
            pltpu.async_copy(acc.at[pl.ds(s * npt + q2, w2)],
                             rows[b].at[pl.ds(0, w2)], gsems[b])


def _msg3_body(h_hbm, ei_hbm, zeros_hbm, out_hbm,
               rows0, gsem0, rows1, gsem1, rows2, gsem2,
               ib0, ism0, ib1, ism1, ib2, ism2,
               ssem0, ssem1, ssem2,
               acc, *, n_pad, e):
    wid, c, s = _worker_id()
    npt = n_pad // 16
    nch = e // CHUNK
    nj = (nch + 31) // 32
    nt = (nj + 2) // 3
    rows = [rows0, rows1, rows2]
    gsems = [gsem0, gsem1, gsem2]
    ssems = [ssem0, ssem1, ssem2]
    pairs = [(ib0, ism0), (ib1, ism1), (ib2, ism2)]

    def idx_start(p, b):
        ib, ism = pairs[b]
        cid = p * 32 + wid

        @pl.when(cid < nch)
        def _():
            pltpu.async_copy(ei_hbm.at[:, pl.ds(cid * CHUNK, CHUNK)], ib, ism)

    def gather_start(p, b):
        ib, ism = pairs[b]
        cid = p * 32 + wid

        @pl.when(cid < nch)
        def _():
            pltpu.make_async_copy(
                ei_hbm.at[:, pl.ds(cid * CHUNK, CHUNK)], ib, ism).wait()
            pltpu.async_copy(h_hbm.at[ib.at[0]], rows[b], gsems[b])

    def finish(p, b):
        ib, _ = pairs[b]
        cid = p * 32 + wid

        @pl.when(cid < nch)
        def _():
            pltpu.make_async_copy(h_hbm.at[ib.at[0]], rows[b], gsems[b]).wait()
            pltpu.async_copy(rows[b], acc.at[ib.at[1]], ssems[b], add=True)

    def drain(p, b):
        ib, _ = pairs[b]
        cid = p * 32 + wid

        @pl.when(cid < nch)
        def _():
            pltpu.make_async_copy(rows[b], acc.at[ib.at[1]], ssems[b]).wait()

    def phase(p, slot, with_drain=True):
        finish(p, slot)
        if with_drain:
            drain(p - 1, (slot + 2) % 3)
        idx_start(p + 2, (slot + 2) % 3)
        gather_start(p + 1, (slot + 1) % 3)

    idx_start(0, 0)
    idx_start(1, 1)
    pltpu.sync_copy(zeros_hbm.at[pl.ds(0, CHUNK)], rows2)
    for q, w in _stripe_chunks(npt):
        pltpu.sync_copy(rows2.at[pl.ds(0, w)], acc.at[pl.ds(s * npt + q, w)])
    plsc.subcore_barrier()
    gather_start(0, 0)

    phase(0, 0, with_drain=False)
    phase(1, 1)

    def body(t, carry):
        p = t * 3 + 2
        phase(p, 2)
        phase(p + 1, 0)
        phase(p + 2, 1)
        return carry

    lax.fori_loop(0, nt, body, 0)

    drain(3 * nt + 1, (3 * nt + 1) % 3)
    plsc.subcore_barrier()

    _writeback(acc, out_hbm, c, s, npt, rows[:2], gsems[:2])


def _sc_mesh():
    return plsc.VectorSubcoreMesh(core_axis_name="c", subcore_axis_name="s")


def _make_msg3_call(n_pad, d, e):
    scratch = []
    for _ in range(3):
        scratch += [pltpu.VMEM((CHUNK, d), jnp.float32), pltpu.SemaphoreType.DMA]
    for _ in range(3):
        scratch += [
            pltpu.VMEM((2, CHUNK), jnp.int32),
            pltpu.SemaphoreType.DMA,
        ]
    scratch += [pltpu.SemaphoreType.DMA] * 3
    scratch.append(pltpu.VMEM_SHARED((n_pad, d), jnp.float32))
    return pl.kernel(
        functools.partial(_msg3_body, n_pad=n_pad, e=e),
        out_type=jax.ShapeDtypeStruct((2, n_pad, d), jnp.float32),
        mesh=_sc_mesh(),
        scratch_types=scratch,
    )


def _make_msg_call(n_pad, d, e, with_deg):
    out_type = [jax.ShapeDtypeStruct((2, n_pad, d), jnp.float32)]
    scratch = [
        pltpu.VMEM((CHUNK, d), jnp.float32),
        pltpu.SemaphoreType.DMA,
        pltpu.VMEM((CHUNK, d), jnp.float32),
        pltpu.SemaphoreType.DMA,
    ]
    for _ in range(4):
        scratch += [
            pltpu.VMEM((2, CHUNK), jnp.int32),
            pltpu.SemaphoreType.DMA,
        ]
    scratch.append(pltpu.VMEM_SHARED((n_pad, d), jnp.float32))
    if with_deg:
        out_type.append(jax.ShapeDtypeStruct((4 * n_pad,), jnp.float32))
        zlen = ((n_pad // 16 + 15) // 16) * 16
        scratch += [
            pltpu.VMEM((CHUNK,), jnp.float32),
            pltpu.VMEM((zlen,), jnp.float32),
            pltpu.VMEM_SHARED((n_pad,), jnp.float32),
            pltpu.VMEM_SHARED((n_pad,), jnp.float32),
        ]
    return pl.kernel(
        functools.partial(_msg_body, n_pad=n_pad, e=e, with_deg=with_deg),
        out_type=out_type,
        mesh=_sc_mesh(),
        scratch_types=scratch,
    )


def _feats_body(degp_ref, w1t_ref, b1_ref, w2t_ref, b2_ref, w3t_ref, b3_ref,
                awt_ref, invd_ref, *, n):
    n_pad = degp_ref.shape[2]
    degs = degp_ref[0] + degp_ref[1]
    deg_in = degs[0:1, :]
    deg_out = degs[1:2, :]
    feats = jnp.concatenate(
        [deg_in, deg_out, jnp.log(1.0 + deg_in), jnp.log(1.0 + deg_out)], axis=0)
    mask = (lax.broadcasted_iota(jnp.int32, (1, n_pad), 1) < n).astype(jnp.float32)
    inv_n = 1.0 / n
    mean = jnp.sum(feats * mask, axis=1, keepdims=True) * inv_n
    var = jnp.sum(((feats - mean) * mask) ** 2, axis=1, keepdims=True) * inv_n
    normed = (feats - mean) / (jnp.sqrt(var) + 1e-6)
    a = jnp.maximum(jnp.dot(w1t_ref[...], normed,
                            preferred_element_type=jnp.float32) + b1_ref[...], 0.0)
    a = jnp.maximum(jnp.dot(w2t_ref[...], a,
                            preferred_element_type=jnp.float32) + b2_ref[...], 0.0)
    logits = jnp.dot(w3t_ref[...], a, preferred_element_type=jnp.float32) + b3_ref[...]
    m = jnp.max(logits, axis=0, keepdims=True)
    ex = jnp.exp(logits - m)
    awt_ref[...] = ex / jnp.sum(ex, axis=0, keepdims=True)
    invd_ref[...] = 1.0 / jnp.maximum(deg_in, 1.0)


def _hop_body(p_ref, invd_ref, h_ref, prim_ref, awk_ref, outp_ref,
              w_ref, wih_ref, whh_ref, bih_ref, bhh_ref,
              hn_ref, outn_ref):
    d = h_ref.shape[1]
    msg = (p_ref[0] + p_ref[1]) * invd_ref[...]
    x = jnp.maximum(jnp.dot(msg, w_ref[...], preferred_element_type=jnp.float32), 0.0)
    gi = jnp.dot(x, wih_ref[...], preferred_element_type=jnp.float32) + bih_ref[...]
    h = h_ref[...]
    gh = jnp.dot(h, whh_ref[...], preferred_element_type=jnp.float32) + bhh_ref[...]
    r = jax.nn.sigmoid(gi[:, 0:d] + gh[:, 0:d])
    z = jax.nn.sigmoid(gi[:, d:2 * d] + gh[:, d:2 * d])
    ng = jnp.tanh(gi[:, 2 * d:3 * d] + r * gh[:, 2 * d:3 * d])
    hn = (1.0 - z) * ng + z * h + prim_ref[...]
    hn_ref[...] = hn
    outn_ref[...] = outp_ref[...] + awk_ref[...] * hn


def _make_hop_call(n, d, rows):
    g = n // rows
    full = lambda i: (0, 0)
    rb = lambda i: (i, 0)
    return pl.pallas_call(
        _hop_body,
        grid=(g,),
        in_specs=[
            pl.BlockSpec((2, rows, d), lambda i: (0, i, 0)),
            pl.BlockSpec((rows, 1), rb),
            pl.BlockSpec((rows, d), rb),
            pl.BlockSpec((rows, d), rb),
            pl.BlockSpec((rows, 1), rb),
            pl.BlockSpec((rows, d), rb),
            pl.BlockSpec((d, d), full),
            pl.BlockSpec((d, 3 * d), full),
            pl.BlockSpec((d, 3 * d), full),
            pl.BlockSpec((1, 3 * d), full),
            pl.BlockSpec((1, 3 * d), full),
        ],
        out_specs=[
            pl.BlockSpec((rows, d), rb),
            pl.BlockSpec((rows, d), rb),
        ],
        out_shape=[
            jax.ShapeDtypeStruct((n, d), jnp.float32),
            jax.ShapeDtypeStruct((n, d), jnp.float32),
        ],
    )


def kernel(edge_index, primary_emb, W_hops, gru_w_ih, gru_w_hh, gru_b_ih,
           gru_b_hh, mlp_w1, mlp_b1, mlp_w2, mlp_b2, mlp_w3, mlp_b3):
    n, d = primary_emb.shape
    e = edge_index.shape[1]
    k_len = W_hops.shape[0]

    n_pad = ((n + 127) // 128) * 128
    zeros_nd = jnp.zeros((n_pad, d), jnp.float32)

    msg_deg_call = _make_msg_call(n_pad, d, e, with_deg=True)
    msg_call = _make_msg3_call(n_pad, d, e)

    partials1, degs = msg_deg_call(primary_emb, edge_index, zeros_nd)
    degp = degs.reshape(2, 2, n_pad)

    feats_call = pl.pallas_call(
        functools.partial(_feats_body, n=n),
        out_shape=[
            jax.ShapeDtypeStruct((k_len, n_pad), jnp.float32),
            jax.ShapeDtypeStruct((1, n_pad), jnp.float32),
        ],
    )
    awt, invd = feats_call(
        degp,
        mlp_w1.T, mlp_b1.reshape(-1, 1),
        mlp_w2.T, mlp_b2.reshape(-1, 1),
        mlp_w3.T, mlp_b3.reshape(-1, 1),
    )
    invd_col = invd[0, :n].reshape(n, 1)
    wih_t = gru_w_ih.T
    whh_t = gru_w_hh.T
    bih_row = gru_b_ih.reshape(1, -1)
    bhh_row = gru_b_hh.reshape(1, -1)

    hop_call = _make_hop_call(n, d, rows=2000)

    h = primary_emb
    out = jnp.zeros((n, d), jnp.float32)
    for hop in range(k_len):
        partials = partials1 if hop == 0 else msg_call(h, edge_index, zeros_nd)
        h, out = hop_call(partials, invd_col, h, primary_emb,
                          awt[hop, :n].reshape(n, 1), out,
                          W_hops[hop], wih_t, whh_t, bih_row, bhh_row)
    return out


# speedup vs baseline: 13.0068x; 1.1156x over previous
import functools

import jax
import jax.numpy as jnp
from jax import lax
from jax.experimental import pallas as pl
from jax.experimental.pallas import tpu as pltpu
from jax.experimental.pallas import tpu_sc as plsc

CHUNK = 128


def _worker_id():
    c = lax.axis_index("c")
    s = lax.axis_index("s")
    return s * 2 + c, c, s


def _stripe_chunks(npt):
    out = []
    q = 0
    while q < npt:
        out.append((q, min(CHUNK, npt - q)))
        q += CHUNK
    return out


def _msg_body(h_hbm, ei_hbm, zeros_hbm, *refs, n_pad, e, with_deg):
    if with_deg:
        (out_hbm, degs_hbm,
         rows0, gsem0, rows1, gsem1,
         ib0, ism0, ib1, ism1, ib2, ism2, ib3, ism3,
         acc, ones_v, zbuf, accd_in, accd_out) = refs
    else:
        (out_hbm,
         rows0, gsem0, rows1, gsem1,
         ib0, ism0, ib1, ism1, ib2, ism2, ib3, ism3,
         acc) = refs
    wid, c, s = _worker_id()
    npt = n_pad // 16
    nch = e // CHUNK
    nj = (nch + 31) // 32
    nt = (nj + 3) // 4
    rows = [rows0, rows1]
    gsems = [gsem0, gsem1]
    pairs = [(ib0, ism0), (ib1, ism1), (ib2, ism2), (ib3, ism3)]

    def idx_start(j, p):
        ib, ism = pairs[p]
        cid = j * 32 + wid

        @pl.when(cid < nch)
        def _():
            pltpu.async_copy(ei_hbm.at[:, pl.ds(cid * CHUNK, CHUNK)], ib, ism)

    idx_start(0, 0)
    idx_start(1, 1)
    idx_start(2, 2)
    idx_start(3, 3)

    pltpu.sync_copy(zeros_hbm.at[pl.ds(0, CHUNK)], rows0)
    for q, w in _stripe_chunks(npt):
        pltpu.sync_copy(rows0.at[pl.ds(0, w)], acc.at[pl.ds(s * npt + q, w)])
    if with_deg:
        for i in range(CHUNK // 16):
            ones_v[pl.ds(i * 16, 16)] = jnp.ones((16,), jnp.float32)
        zlen = zbuf.shape[0]
        for i in range(zlen // 16):
            zbuf[pl.ds(i * 16, 16)] = jnp.zeros((16,), jnp.float32)
        pltpu.sync_copy(zbuf.at[pl.ds(0, npt)], accd_in.at[pl.ds(s * npt, npt)])
        pltpu.sync_copy(zbuf.at[pl.ds(0, npt)], accd_out.at[pl.ds(s * npt, npt)])
    plsc.subcore_barrier()

    def gather_start(j, b, p):
        ib, ism = pairs[p]
        cid = j * 32 + wid

        @pl.when(cid < nch)
        def _():
            pltpu.make_async_copy(
                ei_hbm.at[:, pl.ds(cid * CHUNK, CHUNK)], ib, ism).wait()
            pltpu.async_copy(h_hbm.at[ib.at[0]], rows[b], gsems[b])

    def finish(j, b, p):
        ib, ism = pairs[p]
        cid = j * 32 + wid

        @pl.when(cid < nch)
        def _():
            pltpu.make_async_copy(h_hbm.at[ib.at[0]], rows[b], gsems[b]).wait()
            pltpu.sync_copy(rows[b], acc.at[ib.at[1]], add=True)
            if with_deg:
                pltpu.sync_copy(ones_v, accd_in.at[ib.at[1]], add=True)
                pltpu.sync_copy(ones_v, accd_out.at[ib.at[0]], add=True)

    gather_start(0, 0, 0)
    gather_start(1, 1, 1)

    def body(t, carry):
        j = t * 4
        for off, b in ((0, 0), (1, 1), (2, 0), (3, 1)):
            p = off
            p_next = (off + 2) % 4
            finish(j + off, b, p)
            idx_start(j + off + 4, p)
            gather_start(j + off + 2, b, p_next)
        return carry

    lax.fori_loop(0, nt, body, 0)
    plsc.subcore_barrier()

    _writeback(acc, out_hbm, c, s, npt, rows, gsems)
    if with_deg:
        base_c = c * (2 * n_pad)
        pltpu.sync_copy(accd_in.at[pl.ds(s * npt, npt)], zbuf.at[pl.ds(0, npt)])
        pltpu.sync_copy(zbuf.at[pl.ds(0, npt)],
                        degs_hbm.at[pl.ds(base_c + s * npt, npt)])
        pltpu.sync_copy(accd_out.at[pl.ds(s * npt, npt)], zbuf.at[pl.ds(0, npt)])
        pltpu.sync_copy(zbuf.at[pl.ds(0, npt)],
                        degs_hbm.at[pl.ds(base_c + n_pad + s * npt, npt)])


def _writeback(acc, out_hbm, c, s, npt, rows, gsems):
    chunks = _stripe_chunks(npt)
    nb = len(rows)

    for i, (q, w) in enumerate(chunks):
        if i >= 2:
            continue
        pltpu.async_copy(acc.at[pl.ds(s * npt + q, w)],
                         rows[i % nb].at[pl.ds(0, w)], gsems[i % nb])
    for i, (q, w) in enumerate(chunks):
        b = i % nb
        pltpu.make_async_copy(acc.at[pl.ds(s * npt + q, w)],
                              rows[b].at[pl.ds(0, w)], gsems[b]).wait()
        pltpu.sync_copy(rows[b].at[pl.ds(0, w)], out_hbm.at[c, pl.ds(s * npt + q, w)])
        if i + 2 < len(chunks):
            q2, w2 = chunks[i + 2]
            pltpu.async_copy(acc.at[pl.ds(s * npt + q2, w2)],
                             rows[b].at[pl.ds(0, w2)], gsems[b])


def _msg3_body(h_hbm, ei_hbm, zeros_hbm, out_hbm,
               rows0, gsem0, rows1, gsem1, rows2, gsem2,
               ib0, ism0, ib1, ism1, ib2, ism2,
               acc, *, n_pad, e):
    wid, c, s = _worker_id()
    npt = n_pad // 16
    nch = e // CHUNK
    nj = (nch + 31) // 32
    nt = (nj + 2) // 3
    rows = [rows0, rows1, rows2]
    gsems = [gsem0, gsem1, gsem2]
    pairs = [(ib0, ism0), (ib1, ism1), (ib2, ism2)]

    def idx_start(j, p):
        ib, ism = pairs[p]
        cid = j * 32 + wid

        @pl.when(cid < nch)
        def _():
            pltpu.async_copy(ei_hbm.at[:, pl.ds(cid * CHUNK, CHUNK)], ib, ism)

    idx_start(0, 0)
    idx_start(1, 1)
    idx_start(2, 2)
    pltpu.sync_copy(zeros_hbm.at[pl.ds(0, CHUNK)], rows2)
    for q, w in _stripe_chunks(npt):
        pltpu.sync_copy(rows2.at[pl.ds(0, w)], acc.at[pl.ds(s * npt + q, w)])
    plsc.subcore_barrier()

    def gather_start(j, b):
        ib, ism = pairs[b]
        cid = j * 32 + wid

        @pl.when(cid < nch)
        def _():
            pltpu.make_async_copy(
                ei_hbm.at[:, pl.ds(cid * CHUNK, CHUNK)], ib, ism).wait()
            pltpu.async_copy(h_hbm.at[ib.at[0]], rows[b], gsems[b])

    def finish(j, b):
        ib, ism = pairs[b]
        cid = j * 32 + wid

        @pl.when(cid < nch)
        def _():
            pltpu.make_async_copy(h_hbm.at[ib.at[0]], rows[b], gsems[b]).wait()
            pltpu.sync_copy(rows[b], acc.at[ib.at[1]], add=True)

    gather_start(0, 0)
    gather_start(1, 1)

    def body(t, carry):
        j = t * 3
        for off in (0, 1, 2):
            finish(j + off, off)
            idx_start(j + off + 3, off)
            gather_start(j + off + 2, (off + 2) % 3)
        return carry

    lax.fori_loop(0, nt, body, 0)
    plsc.subcore_barrier()

    _writeback(acc, out_hbm, c, s, npt, rows[:2], gsems[:2])


def _sc_mesh():
    return plsc.VectorSubcoreMesh(core_axis_name="c", subcore_axis_name="s")


def _make_msg3_call(n_pad, d, e):
    scratch = []
    for _ in range(3):
        scratch += [pltpu.VMEM((CHUNK, d), jnp.float32), pltpu.SemaphoreType.DMA]
    for _ in range(3):
        scratch += [
            pltpu.VMEM((2, CHUNK), jnp.int32),
            pltpu.SemaphoreType.DMA,
        ]
    scratch.append(pltpu.VMEM_SHARED((n_pad, d), jnp.float32))
    return pl.kernel(
        functools.partial(_msg3_body, n_pad=n_pad, e=e),
        out_type=jax.ShapeDtypeStruct((2, n_pad, d), jnp.float32),
        mesh=_sc_mesh(),
        scratch_types=scratch,
    )


def _make_msg_call(n_pad, d, e, with_deg):
    out_type = [jax.ShapeDtypeStruct((2, n_pad, d), jnp.float32)]
    scratch = [
        pltpu.VMEM((CHUNK, d), jnp.float32),
        pltpu.SemaphoreType.DMA,
        pltpu.VMEM((CHUNK, d), jnp.float32),
        pltpu.SemaphoreType.DMA,
    ]
    for _ in range(4):
        scratch += [
            pltpu.VMEM((2, CHUNK), jnp.int32),
            pltpu.SemaphoreType.DMA,
        ]
    scratch.append(pltpu.VMEM_SHARED((n_pad, d), jnp.float32))
    if with_deg:
        out_type.append(jax.ShapeDtypeStruct((4 * n_pad,), jnp.float32))
        zlen = ((n_pad // 16 + 15) // 16) * 16
        scratch += [
            pltpu.VMEM((CHUNK,), jnp.float32),
            pltpu.VMEM((zlen,), jnp.float32),
            pltpu.VMEM_SHARED((n_pad,), jnp.float32),
            pltpu.VMEM_SHARED((n_pad,), jnp.float32),
        ]
    return pl.kernel(
        functools.partial(_msg_body, n_pad=n_pad, e=e, with_deg=with_deg),
        out_type=out_type,
        mesh=_sc_mesh(),
        scratch_types=scratch,
    )


def _feats_body(degp_ref, w1t_ref, b1_ref, w2t_ref, b2_ref, w3t_ref, b3_ref,
                awt_ref, invd_ref, *, n):
    n_pad = degp_ref.shape[2]
    degs = degp_ref[0] + degp_ref[1]
    deg_in = degs[0:1, :]
    deg_out = degs[1:2, :]
    feats = jnp.concatenate(
        [deg_in, deg_out, jnp.log(1.0 + deg_in), jnp.log(1.0 + deg_out)], axis=0)
    mask = (lax.broadcasted_iota(jnp.int32, (1, n_pad), 1) < n).astype(jnp.float32)
    inv_n = 1.0 / n
    mean = jnp.sum(feats * mask, axis=1, keepdims=True) * inv_n
    var = jnp.sum(((feats - mean) * mask) ** 2, axis=1, keepdims=True) * inv_n
    normed = (feats - mean) / (jnp.sqrt(var) + 1e-6)
    a = jnp.maximum(jnp.dot(w1t_ref[...], normed,
                            preferred_element_type=jnp.float32) + b1_ref[...], 0.0)
    a = jnp.maximum(jnp.dot(w2t_ref[...], a,
                            preferred_element_type=jnp.float32) + b2_ref[...], 0.0)
    logits = jnp.dot(w3t_ref[...], a, preferred_element_type=jnp.float32) + b3_ref[...]
    m = jnp.max(logits, axis=0, keepdims=True)
    ex = jnp.exp(logits - m)
    awt_ref[...] = ex / jnp.sum(ex, axis=0, keepdims=True)
    invd_ref[...] = 1.0 / jnp.maximum(deg_in, 1.0)


def _hop_body(p_ref, invd_ref, h_ref, prim_ref, awk_ref, outp_ref,
              w_ref, wih_ref, whh_ref, bih_ref, bhh_ref,
              hn_ref, outn_ref):
    d = h_ref.shape[1]
    msg = (p_ref[0] + p_ref[1]) * invd_ref[...]
    x = jnp.maximum(jnp.dot(msg, w_ref[...], preferred_element_type=jnp.float32), 0.0)
    gi = jnp.dot(x, wih_ref[...], preferred_element_type=jnp.float32) + bih_ref[...]
    h = h_ref[...]
    gh = jnp.dot(h, whh_ref[...], preferred_element_type=jnp.float32) + bhh_ref[...]
    r = jax.nn.sigmoid(gi[:, 0:d] + gh[:, 0:d])
    z = jax.nn.sigmoid(gi[:, d:2 * d] + gh[:, d:2 * d])
    ng = jnp.tanh(gi[:, 2 * d:3 * d] + r * gh[:, 2 * d:3 * d])
    hn = (1.0 - z) * ng + z * h + prim_ref[...]
    hn_ref[...] = hn
    outn_ref[...] = outp_ref[...] + awk_ref[...] * hn


def _make_hop_call(n, d, rows):
    g = n // rows
    full = lambda i: (0, 0)
    rb = lambda i: (i, 0)
    return pl.pallas_call(
        _hop_body,
        grid=(g,),
        in_specs=[
            pl.BlockSpec((2, rows, d), lambda i: (0, i, 0)),
            pl.BlockSpec((rows, 1), rb),
            pl.BlockSpec((rows, d), rb),
            pl.BlockSpec((rows, d), rb),
            pl.BlockSpec((rows, 1), rb),
            pl.BlockSpec((rows, d), rb),
            pl.BlockSpec((d, d), full),
            pl.BlockSpec((d, 3 * d), full),
            pl.BlockSpec((d, 3 * d), full),
            pl.BlockSpec((1, 3 * d), full),
            pl.BlockSpec((1, 3 * d), full),
        ],
        out_specs=[
            pl.BlockSpec((rows, d), rb),
            pl.BlockSpec((rows, d), rb),
        ],
        out_shape=[
            jax.ShapeDtypeStruct((n, d), jnp.float32),
            jax.ShapeDtypeStruct((n, d), jnp.float32),
        ],
    )


def kernel(edge_index, primary_emb, W_hops, gru_w_ih, gru_w_hh, gru_b_ih,
           gru_b_hh, mlp_w1, mlp_b1, mlp_w2, mlp_b2, mlp_w3, mlp_b3):
    n, d = primary_emb.shape
    e = edge_index.shape[1]
    k_len = W_hops.shape[0]

    n_pad = ((n + 127) // 128) * 128
    zeros_nd = jnp.zeros((n_pad, d), jnp.float32)

    msg_deg_call = _make_msg_call(n_pad, d, e, with_deg=True)
    msg_call = _make_msg3_call(n_pad, d, e)

    partials1, degs = msg_deg_call(primary_emb, edge_index, zeros_nd)
    degp = degs.reshape(2, 2, n_pad)

    feats_call = pl.pallas_call(
        functools.partial(_feats_body, n=n),
        out_shape=[
            jax.ShapeDtypeStruct((k_len, n_pad), jnp.float32),
            jax.ShapeDtypeStruct((1, n_pad), jnp.float32),
        ],
    )
    awt, invd = feats_call(
        degp,
        mlp_w1.T, mlp_b1.reshape(-1, 1),
        mlp_w2.T, mlp_b2.reshape(-1, 1),
        mlp_w3.T, mlp_b3.reshape(-1, 1),
    )
    invd_col = invd[0, :n].reshape(n, 1)
    wih_t = gru_w_ih.T
    whh_t = gru_w_hh.T
    bih_row = gru_b_ih.reshape(1, -1)
    bhh_row = gru_b_hh.reshape(1, -1)

    hop_call = _make_hop_call(n, d, rows=2000)

    h = primary_emb
    out = jnp.zeros((n, d), jnp.float32)
    for hop in range(k_len):
        partials = partials1 if hop == 0 else msg_call(h, edge_index, zeros_nd)
        h, out = hop_call(partials, invd_col, h, primary_emb,
                          awt[hop, :n].reshape(n, 1), out,
                          W_hops[hop], wih_t, whh_t, bih_row, bhh_row)
    return out
